# baseline (speedup 1.0000x reference)
import functools

import jax
import jax.numpy as jnp
from jax import lax
from jax.experimental import pallas as pl
from jax.experimental.pallas import tpu as pltpu
from jax.experimental.pallas import tpu_sc as plsc

NC = 2
NS = 16
LANES = 16
NW = NC * NS


def _pad_up(n, m):
    return (n + m - 1) // m * m


def _sc_degree_kernel(ept, n_pad, rows_per_tile):
    mesh = plsc.VectorSubcoreMesh(core_axis_name="c", subcore_axis_name="s")

    @functools.partial(
        pl.kernel,
        out_type=jax.ShapeDtypeStruct((NC, n_pad), jnp.float32),
        mesh=mesh,
        scratch_types=[
            pltpu.VMEM((ept,), jnp.int32),
            pltpu.VMEM((ept,), jnp.float32),
            pltpu.VMEM((rows_per_tile,), jnp.float32),
            pltpu.VMEM_SHARED((n_pad,), jnp.float32),
        ],
    )
    def deg_kernel(dst_hbm, out_hbm, idx_v, ones_v, zero_v, deg_sh):
        cid = lax.axis_index("c")
        sid = lax.axis_index("s")
        base = sid * rows_per_tile

        def fill_zero(r, carry):
            zero_v[pl.ds(r * LANES, LANES)] = jnp.zeros((LANES,), jnp.float32)
            return carry

        lax.fori_loop(0, rows_per_tile // LANES, fill_zero, 0)
        pltpu.sync_copy(zero_v, deg_sh.at[pl.ds(base, rows_per_tile)])

        def fill_one(r, carry):
            ones_v[pl.ds(r * LANES, LANES)] = jnp.ones((LANES,), jnp.float32)
            return carry

        lax.fori_loop(0, ept // LANES, fill_one, 0)
        pltpu.sync_copy(dst_hbm.at[cid, sid], idx_v)
        plsc.subcore_barrier()
        pltpu.sync_copy(ones_v, deg_sh.at[idx_v], add=True)
        plsc.subcore_barrier()
        pltpu.sync_copy(
            deg_sh.at[pl.ds(base, rows_per_tile)],
            out_hbm.at[cid, pl.ds(base, rows_per_tile)],
        )

    return deg_kernel


def _sc_aggregate_kernel(n_mega, chunk, n_pad, rows_per_tile, d):
    mesh = plsc.VectorSubcoreMesh(core_axis_name="c", subcore_axis_name="s")

    @functools.partial(
        pl.kernel,
        out_type=jax.ShapeDtypeStruct((NC, n_pad, d), jnp.float32),
        mesh=mesh,
        scratch_types=[
            pltpu.VMEM((n_mega, chunk), jnp.int32),
            pltpu.VMEM((n_mega, chunk), jnp.int32),
            pltpu.VMEM((chunk, d), jnp.float32),
            pltpu.VMEM((chunk, d), jnp.float32),
            pltpu.VMEM((rows_per_tile, d), jnp.float32),
            pltpu.VMEM_SHARED((n_pad, d), jnp.float32),
            pltpu.SemaphoreType.DMA,
            pltpu.SemaphoreType.DMA,
            pltpu.SemaphoreType.DMA,
            pltpu.SemaphoreType.DMA,
        ],
        compiler_params=pltpu.CompilerParams(use_tc_tiling_on_sc=False),
    )
    def agg_kernel(src_hbm, dst_hbm, g_hbm, out_hbm,
                   sidx_v, didx_v, rows0_v, rows1_v, zero_v, acc_sh,
                   sg0, sg1, ss0, ss1):
        cid = lax.axis_index("c")
        sid = lax.axis_index("s")
        base = sid * rows_per_tile

        def fill_zero(r, carry):
            zero_v[r, :] = jnp.zeros((LANES,), jnp.float32)
            return carry

        lax.fori_loop(0, rows_per_tile, fill_zero, 0)
        pltpu.sync_copy(zero_v, acc_sh.at[pl.ds(base, rows_per_tile)])
        pltpu.sync_copy(src_hbm.at[cid, sid], sidx_v)
        pltpu.sync_copy(dst_hbm.at[cid, sid], didx_v)
        plsc.subcore_barrier()

        rows = (rows0_v, rows1_v)
        sgs = (sg0, sg1)
        sss = (ss0, ss1)

        gat = [None, None]
        sca = [None, None]
        gat[0] = pltpu.async_copy(g_hbm.at[sidx_v.at[0]], rows[0], sgs[0])
        for m in range(n_mega):
            b = m & 1
            if m + 1 < n_mega:
                b2 = (m + 1) & 1
                if sca[b2] is not None:
                    sca[b2].wait()
                gat[b2] = pltpu.async_copy(
                    g_hbm.at[sidx_v.at[m + 1]], rows[b2], sgs[b2])
            gat[b].wait()
            sca[b] = pltpu.async_copy(
                rows[b], acc_sh.at[didx_v.at[m]], sss[b], add=True)
        for b in range(2):
            if sca[b] is not None:
                sca[b].wait()
        plsc.subcore_barrier()
        pltpu.sync_copy(
            acc_sh.at[pl.ds(base, rows_per_tile)],
            out_hbm.at[cid, pl.ds(base, rows_per_tile)],
        )

    return agg_kernel


def _tc_matmul_scale(x, w1, degt):
    n, k = x.shape
    d = w1.shape[1]
    blk = 2000 if n % 2000 == 0 else n
    grid = n // blk

    def mm_kernel(x_ref, w_ref, deg_ref, o_ref):
        deg = deg_ref[:, 0] + deg_ref[:, 1] + 1.0
        dinv = lax.rsqrt(deg)
        h = jnp.dot(x_ref[...], w_ref[...], preferred_element_type=jnp.float32)
        o_ref[...] = h * dinv[:, None]

    return pl.pallas_call(
        mm_kernel,
        grid=(grid,),
        in_specs=[
            pl.BlockSpec((blk, k), lambda i: (i, 0)),
            pl.BlockSpec((k, d), lambda i: (0, 0)),
            pl.BlockSpec((blk, 2), lambda i: (i, 0)),
        ],
        out_specs=pl.BlockSpec((blk, d), lambda i: (i, 0)),
        out_shape=jax.ShapeDtypeStruct((n, d), jnp.float32),
    )(x, w1, degt)


def _tc_final(accp, g, degt, b1, w2, b2):
    n, d = g.shape
    blk = 2000 if n % 2000 == 0 else n
    grid = n // blk

    def fin_kernel(acc_ref, g_ref, deg_ref, b1_ref, w2_ref, b2_ref, o_ref):
        deg = deg_ref[:, 0] + deg_ref[:, 1] + 1.0
        dinv = lax.rsqrt(deg)
        tot = acc_ref[0] + acc_ref[1] + g_ref[...]
        agg = tot * dinv[:, None] + b1_ref[0, :]
        h1 = jnp.maximum(agg, 0.0)
        o_ref[...] = (
            jnp.dot(h1, w2_ref[...], preferred_element_type=jnp.float32)
            + b2_ref[0, :]
        )

    return pl.pallas_call(
        fin_kernel,
        grid=(grid,),
        in_specs=[
            pl.BlockSpec((2, blk, d), lambda i: (0, i, 0)),
            pl.BlockSpec((blk, d), lambda i: (i, 0)),
            pl.BlockSpec((blk, 2), lambda i: (i, 0)),
            pl.BlockSpec((1, d), lambda i: (0, 0)),
            pl.BlockSpec((d, d), lambda i: (0, 0)),
            pl.BlockSpec((1, d), lambda i: (0, 0)),
        ],
        out_specs=pl.BlockSpec((blk, d), lambda i: (i, 0)),
        out_shape=jax.ShapeDtypeStruct((n, d), jnp.float32),
    )(accp, g, degt, b1, w2, b2)


def kernel(x, edge_index, W1, b1, W2, b2):
    n, k_in = x.shape
    d = W1.shape[1]
    e = edge_index.shape[1]

    n_pad = _pad_up(n + 1, 128 * NS)
    rows_per_tile = n_pad // NS
    ept_raw = -(-e // NW)
    n_mega = -(-ept_raw // 2048)
    chunk = _pad_up(-(-ept_raw // n_mega), 8)
    ept = n_mega * chunk
    e_pad = NW * ept
    pad = e_pad - e

    src = edge_index[0].astype(jnp.int32)
    dst = edge_index[1].astype(jnp.int32)
    if pad:
        trash = n + jnp.arange(pad, dtype=jnp.int32) % (n_pad - n)
        src = jnp.concatenate([src, jnp.zeros((pad,), jnp.int32)])
        dst = jnp.concatenate([dst, trash])
    src_r = src.reshape(NC, NS, n_mega, chunk)
    dst_r = dst.reshape(NC, NS, n_mega, chunk)
    dst_flat = dst_r.reshape(NC, NS, ept)

    degp = _sc_degree_kernel(ept, n_pad, rows_per_tile)(dst_flat)
    degt = degp.T

    g = _tc_matmul_scale(x, W1, degt)

    accp = _sc_aggregate_kernel(n_mega, chunk, n_pad, rows_per_tile, d)(
        src_r, dst_r, g)

    return _tc_final(accp, g, degt, b1.reshape(1, d), W2, b2.reshape(1, d))


# speedup vs baseline: 1.0562x; 1.0562x over previous
import functools

import jax
import jax.numpy as jnp
from jax import lax
from jax.experimental import pallas as pl
from jax.experimental.pallas import tpu as pltpu
from jax.experimental.pallas import tpu_sc as plsc

NC = 2
NS = 16
LANES = 16
NW = NC * NS


def _pad_up(n, m):
    return (n + m - 1) // m * m


def _sc_degree_kernel(ept, n_pad, rows_per_tile):
    mesh = plsc.VectorSubcoreMesh(core_axis_name="c", subcore_axis_name="s")

    @functools.partial(
        pl.kernel,
        out_type=jax.ShapeDtypeStruct((NC, n_pad), jnp.float32),
        mesh=mesh,
        scratch_types=[
            pltpu.VMEM((ept,), jnp.int32),
            pltpu.VMEM((ept,), jnp.float32),
            pltpu.VMEM((rows_per_tile,), jnp.float32),
            pltpu.VMEM_SHARED((n_pad,), jnp.float32),
        ],
    )
    def deg_kernel(dst_hbm, out_hbm, idx_v, ones_v, zero_v, deg_sh):
        cid = lax.axis_index("c")
        sid = lax.axis_index("s")
        wid = cid * NS + sid
        base = sid * rows_per_tile

        def fill_zero(r, carry):
            zero_v[pl.ds(r * LANES, LANES)] = jnp.zeros((LANES,), jnp.float32)
            return carry

        lax.fori_loop(0, rows_per_tile // LANES, fill_zero, 0)
        pltpu.sync_copy(zero_v, deg_sh.at[pl.ds(base, rows_per_tile)])

        def fill_one(r, carry):
            ones_v[pl.ds(r * LANES, LANES)] = jnp.ones((LANES,), jnp.float32)
            return carry

        lax.fori_loop(0, ept // LANES, fill_one, 0)
        pltpu.sync_copy(dst_hbm.at[pl.ds(wid * ept, ept)], idx_v)
        plsc.subcore_barrier()
        pltpu.sync_copy(ones_v, deg_sh.at[idx_v], add=True)
        plsc.subcore_barrier()
        pltpu.sync_copy(
            deg_sh.at[pl.ds(base, rows_per_tile)],
            out_hbm.at[cid, pl.ds(base, rows_per_tile)],
        )

    return deg_kernel


def _sc_aggregate_kernel(n_mega, chunk, n_pad, rows_per_tile, d):
    mesh = plsc.VectorSubcoreMesh(core_axis_name="c", subcore_axis_name="s")

    @functools.partial(
        pl.kernel,
        out_type=jax.ShapeDtypeStruct((NC, n_pad, d), jnp.float32),
        mesh=mesh,
        scratch_types=[
            pltpu.VMEM((n_mega, chunk), jnp.int32),
            pltpu.VMEM((n_mega, chunk), jnp.int32),
            pltpu.VMEM((chunk, d), jnp.float32),
            pltpu.VMEM((chunk, d), jnp.float32),
            pltpu.VMEM((rows_per_tile, d), jnp.float32),
            pltpu.VMEM_SHARED((n_pad, d), jnp.float32),
            pltpu.SemaphoreType.DMA,
            pltpu.SemaphoreType.DMA,
            pltpu.SemaphoreType.DMA,
            pltpu.SemaphoreType.DMA,
        ],
        compiler_params=pltpu.CompilerParams(use_tc_tiling_on_sc=False),
    )
    def agg_kernel(src_hbm, dst_hbm, g_hbm, out_hbm,
                   sidx_v, didx_v, rows0_v, rows1_v, zero_v, acc_sh,
                   sg0, sg1, ss0, ss1):
        cid = lax.axis_index("c")
        sid = lax.axis_index("s")
        wid = cid * NS + sid
        base = sid * rows_per_tile

        def fill_zero(r, carry):
            zero_v[r, :] = jnp.zeros((LANES,), jnp.float32)
            return carry

        lax.fori_loop(0, rows_per_tile, fill_zero, 0)
        pltpu.sync_copy(zero_v, acc_sh.at[pl.ds(base, rows_per_tile)])
        ept = n_mega * chunk
        for m in range(n_mega):
            pltpu.sync_copy(
                src_hbm.at[pl.ds(wid * ept + m * chunk, chunk)], sidx_v.at[m])
            pltpu.sync_copy(
                dst_hbm.at[pl.ds(wid * ept + m * chunk, chunk)], didx_v.at[m])
        plsc.subcore_barrier()

        rows = (rows0_v, rows1_v)
        sgs = (sg0, sg1)
        sss = (ss0, ss1)

        gat = [None, None]
        sca = [None, None]
        gat[0] = pltpu.async_copy(g_hbm.at[sidx_v.at[0]], rows[0], sgs[0])
        for m in range(n_mega):
            b = m & 1
            if m + 1 < n_mega:
                b2 = (m + 1) & 1
                if sca[b2] is not None:
                    sca[b2].wait()
                gat[b2] = pltpu.async_copy(
                    g_hbm.at[sidx_v.at[m + 1]], rows[b2], sgs[b2])
            gat[b].wait()
            sca[b] = pltpu.async_copy(
                rows[b], acc_sh.at[didx_v.at[m]], sss[b], add=True)
        for b in range(2):
            if sca[b] is not None:
                sca[b].wait()
        plsc.subcore_barrier()
        pltpu.sync_copy(
            acc_sh.at[pl.ds(base, rows_per_tile)],
            out_hbm.at[cid, pl.ds(base, rows_per_tile)],
        )

    return agg_kernel


def _tc_matmul_scale(x, w1, degp):
    n, k = x.shape
    d = w1.shape[1]
    blk = 2560 if n % 2560 == 0 else n
    grid = n // blk

    def mm_kernel(x_ref, w_ref, deg_ref, o_ref):
        deg = deg_ref[0, :] + deg_ref[1, :] + 1.0
        dinv = lax.rsqrt(deg)
        h = jnp.dot(x_ref[...], w_ref[...], preferred_element_type=jnp.float32)
        o_ref[...] = h * dinv[:, None]

    return pl.pallas_call(
        mm_kernel,
        grid=(grid,),
        in_specs=[
            pl.BlockSpec((blk, k), lambda i: (i, 0)),
            pl.BlockSpec((k, d), lambda i: (0, 0)),
            pl.BlockSpec((2, blk), lambda i: (0, i)),
        ],
        out_specs=pl.BlockSpec((blk, d), lambda i: (i, 0)),
        out_shape=jax.ShapeDtypeStruct((n, d), jnp.float32),
    )(x, w1, degp)


def _tc_final(accp, g, degp, b1, w2, b2):
    n, d = g.shape
    blk = 2560 if n % 2560 == 0 else n
    grid = n // blk

    def fin_kernel(acc_ref, g_ref, deg_ref, b1_ref, w2_ref, b2_ref, o_ref):
        deg = deg_ref[0, :] + deg_ref[1, :] + 1.0
        dinv = lax.rsqrt(deg)
        tot = acc_ref[0] + acc_ref[1] + g_ref[...]
        agg = tot * dinv[:, None] + b1_ref[0, :]
        h1 = jnp.maximum(agg, 0.0)
        o_ref[...] = (
            jnp.dot(h1, w2_ref[...], preferred_element_type=jnp.float32)
            + b2_ref[0, :]
        )

    return pl.pallas_call(
        fin_kernel,
        grid=(grid,),
        in_specs=[
            pl.BlockSpec((2, blk, d), lambda i: (0, i, 0)),
            pl.BlockSpec((blk, d), lambda i: (i, 0)),
            pl.BlockSpec((2, blk), lambda i: (0, i)),
            pl.BlockSpec((1, d), lambda i: (0, 0)),
            pl.BlockSpec((d, d), lambda i: (0, 0)),
            pl.BlockSpec((1, d), lambda i: (0, 0)),
        ],
        out_specs=pl.BlockSpec((blk, d), lambda i: (i, 0)),
        out_shape=jax.ShapeDtypeStruct((n, d), jnp.float32),
    )(accp, g, degp, b1, w2, b2)


def kernel(x, edge_index, W1, b1, W2, b2):
    n, k_in = x.shape
    d = W1.shape[1]
    e = edge_index.shape[1]

    n_pad = _pad_up(n + 1, 128 * NS)
    rows_per_tile = n_pad // NS
    ept_raw = -(-e // NW)
    n_mega = -(-ept_raw // 2048)
    chunk = _pad_up(-(-ept_raw // n_mega), 8)
    ept = n_mega * chunk
    e_pad = NW * ept
    pad = e_pad - e

    src = edge_index[0].astype(jnp.int32)
    dst = edge_index[1].astype(jnp.int32)
    if pad:
        trash = n + jnp.arange(pad, dtype=jnp.int32) % (n_pad - n)
        src = jnp.concatenate([src, jnp.zeros((pad,), jnp.int32)])
        dst = jnp.concatenate([dst, trash])

    degp = _sc_degree_kernel(ept, n_pad, rows_per_tile)(dst)

    xp = jnp.pad(x, ((0, n_pad - n), (0, 0)))
    g = _tc_matmul_scale(xp, W1, degp)

    accp = _sc_aggregate_kernel(n_mega, chunk, n_pad, rows_per_tile, d)(
        src, dst, g)

    return _tc_final(accp, g, degp, b1.reshape(1, d), W2, b2.reshape(1, d))[:n]
